# baseline (speedup 1.0000x reference)
import jax
import jax.numpy as jnp
from jax import lax
from jax.experimental import pallas as pl
from jax.experimental.pallas import tpu as pltpu
from jax.experimental.pallas import tpu_sc as plsc

NUM_EMB = 1000000
DIM = 64
B_ROWS = 16384
HIST = 20
TOTAL = B_ROWS * HIST

_info = plsc.get_sparse_core_info()
NC, NS = _info.num_cores, _info.num_subcores
NW = NC * NS
ROWS_W = B_ROWS // NW
CH_ROWS = 16
N_CHUNK = ROWS_W // CH_ROWS
CH = CH_ROWS * HIST
N_GRP = CH // 16


def _sc_body(inp_hbm, tab_hbm, s_hbm, out_hbm,
             slab_v, idx4_v, sel_v, rows_v, out_v, s_v, sem):
  wid = lax.axis_index("s") * NC + lax.axis_index("c")
  iota = lax.iota(jnp.int32, 16)
  iota64 = iota * 64
  pltpu.sync_copy(s_hbm, s_v)

  def chunk_body(c, carry):
    row0 = wid * ROWS_W + c * CH_ROWS
    pltpu.sync_copy(inp_hbm.at[pl.ds(row0, CH_ROWS), :], slab_v)

    def srow(r, carry2):
      lo = slab_v[r, pl.ds(0, 16)]
      hi = slab_v[r, pl.ds(4, 16)]
      idx4_v[pl.ds(r * HIST, 16)] = lo >> 3
      sel_v[pl.ds(r * HIST, 16)] = (lo & 7) * 16
      idx4_v[pl.ds(r * HIST + 4, 16)] = hi >> 3
      sel_v[pl.ds(r * HIST + 4, 16)] = (hi & 7) * 16
      return carry2

    lax.fori_loop(0, CH_ROWS, srow, 0)

    copies = [(0, 128), (128, 128), (256, 64)]
    for st, ln in copies:
      pltpu.async_copy(
          tab_hbm.at[idx4_v.at[pl.ds(st, ln)]],
          rows_v.at[pl.ds(st, ln)],
          sem,
      )
    for st, ln in copies:
      pltpu.make_async_copy(
          tab_hbm.at[idx4_v.at[pl.ds(st, ln)]],
          rows_v.at[pl.ds(st, ln)],
          sem,
      ).wait()

    def grp_body(g, carry3):
      rows16 = g * 16 + iota
      sel16 = sel_v[pl.ds(g * 16, 16)]
      obase = g * 1024 + iota64
      for q in range(16):
        wq = plsc.load_gather(rows_v, [rows16, sel16 + q])
        e0 = lax.shift_right_arithmetic(lax.shift_left(wq, 24), 24)
        e1 = lax.shift_right_arithmetic(lax.shift_left(wq, 16), 24)
        e2 = lax.shift_right_arithmetic(lax.shift_left(wq, 8), 24)
        e3 = lax.shift_right_arithmetic(wq, 24)
        f0 = e0.astype(jnp.float32) * s_v[4 * q, :]
        f1 = e1.astype(jnp.float32) * s_v[4 * q + 1, :]
        f2 = e2.astype(jnp.float32) * s_v[4 * q + 2, :]
        f3 = e3.astype(jnp.float32) * s_v[4 * q + 3, :]
        plsc.store_scatter(out_v, [obase + (4 * q)], f0)
        plsc.store_scatter(out_v, [obase + (4 * q + 1)], f1)
        plsc.store_scatter(out_v, [obase + (4 * q + 2)], f2)
        plsc.store_scatter(out_v, [obase + (4 * q + 3)], f3)
      return carry3

    lax.fori_loop(0, N_GRP, grp_body, 0)

    pltpu.sync_copy(
        out_v, out_hbm.at[pl.ds((wid * ROWS_W + c * CH_ROWS) * HIST * DIM,
                                CH * DIM)]
    )
    return carry

  lax.fori_loop(0, N_CHUNK, chunk_body, 0)


def _lookup(inp, table32, s64):
  mesh = plsc.VectorSubcoreMesh(core_axis_name="c", subcore_axis_name="s")
  k = pl.kernel(
      _sc_body,
      mesh=mesh,
      out_type=jax.ShapeDtypeStruct((TOTAL * DIM,), jnp.float32),
      scratch_types=[
          pltpu.VMEM((CH_ROWS, HIST), jnp.int32),
          pltpu.VMEM((CH,), jnp.int32),
          pltpu.VMEM((CH,), jnp.int32),
          pltpu.VMEM((CH, 128), jnp.int32),
          pltpu.VMEM((CH * DIM,), jnp.float32),
          pltpu.VMEM((DIM, 16), jnp.float32),
          pltpu.SemaphoreType.DMA,
      ],
      compiler_params=pltpu.CompilerParams(needs_layout_passes=False),
  )
  return k(inp, table32, s64)


def kernel(input, weight, weight_scaler):
  b, h = input.shape
  table32 = lax.bitcast_convert_type(
      weight.reshape(NUM_EMB // 8, 128, 4), jnp.int32
  )
  s64 = jnp.broadcast_to(
      weight_scaler.astype(jnp.float32).reshape(DIM, 1), (DIM, 16)
  )
  outf = _lookup(input.astype(jnp.int32), table32, s64)
  return outf.astype(jnp.bfloat16).reshape(b, h, DIM)


# speedup vs baseline: 8.5664x; 8.5664x over previous
import jax
import jax.numpy as jnp
from jax import lax
from jax.experimental import pallas as pl
from jax.experimental.pallas import tpu as pltpu
from jax.experimental.pallas import tpu_sc as plsc

NUM_EMB = 1000000
DIM = 64
TOTAL = 16384 * 20

_info = plsc.get_sparse_core_info()
NC, NS = _info.num_cores, _info.num_subcores
NW = NC * NS
PER_W = TOTAL // NW
CHUNK = 2048
N_CHUNK = PER_W // CHUNK
SUB = 128
N_SUB = CHUNK // SUB


def _sc_gather(idx_hbm, table_hbm, out_hbm, idx_v, rows_v, sem):
  wid = lax.axis_index("s") * NC + lax.axis_index("c")

  def body(c, carry):
    base = wid * PER_W + c * CHUNK
    pltpu.sync_copy(idx_hbm.at[wid, pl.ds(c * N_SUB, N_SUB)], idx_v)
    for j in range(N_SUB):
      pltpu.async_copy(
          table_hbm.at[idx_v.at[j]],
          rows_v.at[pl.ds(j * SUB, SUB)],
          sem,
      )
    for j in range(N_SUB):
      pltpu.make_async_copy(
          table_hbm.at[idx_v.at[j]],
          rows_v.at[pl.ds(j * SUB, SUB)],
          sem,
      ).wait()
    pltpu.sync_copy(rows_v, out_hbm.at[pl.ds(base, CHUNK)])
    return carry

  lax.fori_loop(0, N_CHUNK, body, 0)


def _gather_rows(idx, table):
  mesh = plsc.VectorSubcoreMesh(core_axis_name="c", subcore_axis_name="s")
  k = pl.kernel(
      _sc_gather,
      mesh=mesh,
      out_type=jax.ShapeDtypeStruct((TOTAL, DIM), jnp.int8),
      scratch_types=[
          pltpu.VMEM((N_SUB, SUB), jnp.int32),
          pltpu.VMEM((CHUNK, DIM), jnp.int8),
          pltpu.SemaphoreType.DMA,
      ],
      compiler_params=pltpu.CompilerParams(use_tc_tiling_on_sc=False),
  )
  return k(idx, table)


def _dequant_body(x_ref, s_ref, o_ref):
  s = jnp.reshape(s_ref[0:1, 0:DIM], (1, 1, DIM))
  o_ref[...] = (x_ref[...] * s).astype(jnp.bfloat16)


def _dequant(rows, scaler, b, h):
  x = rows.reshape(b, h, DIM)
  s = jnp.broadcast_to(
      jnp.pad(scaler.astype(jnp.float32), (0, 2 * DIM - DIM)).reshape(1, -1),
      (8, 2 * DIM),
  )
  blk = 1024
  return pl.pallas_call(
      _dequant_body,
      grid=(b // blk,),
      in_specs=[
          pl.BlockSpec((blk, h, DIM), lambda i: (i, 0, 0)),
          pl.BlockSpec((8, 2 * DIM), lambda i: (0, 0)),
      ],
      out_specs=pl.BlockSpec((blk, h, DIM), lambda i: (i, 0, 0)),
      out_shape=jax.ShapeDtypeStruct((b, h, DIM), jnp.bfloat16),
  )(x, s)


def kernel(input, weight, weight_scaler):
  b, h = input.shape
  idx = input.reshape(NW, PER_W // SUB, SUB).astype(jnp.int32)
  rows = _gather_rows(idx, weight)
  return _dequant(rows, weight_scaler, b, h)
